# baseline (speedup 1.0000x reference)
import functools
import math

import jax
import jax.numpy as jnp
from jax import lax
from jax.experimental import pallas as pl
from jax.experimental.pallas import tpu as pltpu
from jax.experimental.pallas import tpu_sc as plsc

_V = 1024
_D = 256
_NF = 10
_W = _D + 2 * _NF
_B = 16384
_F = 3
_NW = 32
_ROWS_W = _B // _NW
_R = 32
_NCHUNK = _ROWS_W // _R
_NBUF = 3


def _fourier(rows):
    d = lax.broadcasted_iota(jnp.int32, (rows, 2 * _NF), 0).astype(jnp.float32)
    k = lax.broadcasted_iota(jnp.int32, (rows, 2 * _NF), 1)
    kk = k % _NF
    coef = (jnp.float32(math.pi) / jnp.float32(_V)) * (
        lax.shift_left(jnp.int32(1), kk).astype(jnp.float32))
    raw = coef * d
    return jnp.where(k < _NF, jnp.sin(raw), jnp.cos(raw))


def _tables_body(emb_ref, tb_ref, tc_ref, ftab_ref):
    four = _fourier(_V)
    tb_ref[...] = jnp.concatenate(
        [jnp.zeros((_V, 2 * _NF), jnp.float32), emb_ref[:, :_D - 2 * _NF]],
        axis=1)
    tc_ref[...] = jnp.concatenate(
        [emb_ref[:, _D - 2 * _NF:], four,
         jnp.zeros((_V, 128 - 4 * _NF), jnp.float32)], axis=1)
    dr = lax.broadcasted_iota(jnp.int32, (_V // 4, 128), 0)
    c = lax.broadcasted_iota(jnp.int32, (_V // 4, 128), 1)
    d = (4 * dr + c // 32).astype(jnp.float32)
    k = c % 32
    kk = k % _NF
    coef = (jnp.float32(math.pi) / jnp.float32(_V)) * (
        lax.shift_left(jnp.int32(1), kk).astype(jnp.float32))
    raw = coef * d
    val = jnp.where(k < _NF, jnp.sin(raw),
                    jnp.where(k < 2 * _NF, jnp.cos(raw), 0.0))
    ftab_ref[...] = val


def _build_tables(embed_table):
    return pl.pallas_call(
        _tables_body,
        out_shape=(
            jax.ShapeDtypeStruct((_V, _D), jnp.float32),
            jax.ShapeDtypeStruct((_V, 128), jnp.float32),
            jax.ShapeDtypeStruct((_V // 4, 128), jnp.float32),
        ),
    )(embed_table)


def _sc_body(emb, tb, tc, ftab, t_flat, out, t_v, idx_e, idx_o, ftab_v,
             obufs, cbufs, sas, sbs, scs):
    wid = lax.axis_index("s") * 2 + lax.axis_index("c")
    pltpu.sync_copy(t_flat.at[pl.ds(wid * (_ROWS_W * _F), _ROWS_W * _F)], t_v)
    pltpu.sync_copy(ftab, ftab_v)

    lane = lax.iota(jnp.int32, 16)
    zero = lane * 0
    for u in range(_ROWS_W // 16):
        b = lane + (u * 16)
        lo = b * _F
        t0 = plsc.load_gather(t_v, [lo])
        t1 = plsc.load_gather(t_v, [lo + 1])
        t2 = plsc.load_gather(t_v, [lo + 2])
        cc = u // (_R // 16)
        off = (u % (_R // 16)) * 16
        idx_e[cc, pl.ds(off, 16)] = t1 - t0
        idx_o[cc, pl.ds(off, 16)] = t2 - t1

    orow_base = wid * _ROWS_W

    def _fire(c):
        p = c % _NBUF
        ga = pltpu.async_copy(
            emb.at[idx_e.at[c]], obufs[p].at[:, pl.ds(0, _D)], sas[p])
        gb = pltpu.async_copy(
            tb.at[idx_o.at[c]], obufs[p].at[:, pl.ds(_D, _D)], sbs[p])
        gc = pltpu.async_copy(tc.at[idx_o.at[c]], cbufs[p], scs[p])
        return (ga, gb, gc)

    def _repair(c):
        p = c % _NBUF
        obuf = obufs[p]
        cbuf = cbufs[p]
        for s in range(_R // 16):
            rows = lane + (s * 16)
            d0v = idx_e[c, pl.ds(s * 16, 16)]
            frv = d0v >> 2
            fcv = (d0v & 3) * 32

            def f_fix(k, _):
                vals = plsc.load_gather(ftab_v, [frv, fcv + k])
                plsc.store_scatter(obuf, [rows, zero + (_D + k)], vals)
                return 0

            def t_fix(k, _):
                vals = plsc.load_gather(cbuf, [rows, zero + k])
                plsc.store_scatter(obuf, [rows, zero + (2 * _D + k)], vals)
                return 0

            lax.fori_loop(0, 2 * _NF, f_fix, 0)
            lax.fori_loop(0, 4 * _NF, t_fix, 0)

    gh = [None] * _NCHUNK
    for c in range(_NBUF - 1):
        gh[c] = _fire(c)
    for c in range(_NCHUNK):
        if c + _NBUF - 1 < _NCHUNK:
            gh[c + _NBUF - 1] = _fire(c + _NBUF - 1)
        for h in gh[c]:
            h.wait()
        _repair(c)
        pltpu.sync_copy(obufs[c % _NBUF],
                        out.at[pl.ds(orow_base + c * _R, _R)])


@functools.partial(
    pl.kernel,
    out_type=jax.ShapeDtypeStruct((_B, 2 * _W), jnp.float32),
    mesh=plsc.VectorSubcoreMesh(core_axis_name="c", subcore_axis_name="s"),
    compiler_params=pltpu.CompilerParams(needs_layout_passes=False),
    scratch_types=[
        pltpu.VMEM((_ROWS_W * _F,), jnp.int32),
        pltpu.VMEM((_NCHUNK, _R), jnp.int32),
        pltpu.VMEM((_NCHUNK, _R), jnp.int32),
        pltpu.VMEM((_V // 4, 128), jnp.float32),
        pltpu.VMEM((_R, 2 * _W), jnp.float32),
        pltpu.VMEM((_R, 2 * _W), jnp.float32),
        pltpu.VMEM((_R, 2 * _W), jnp.float32),
        pltpu.VMEM((_R, 128), jnp.float32),
        pltpu.VMEM((_R, 128), jnp.float32),
        pltpu.VMEM((_R, 128), jnp.float32),
        pltpu.SemaphoreType.DMA,
        pltpu.SemaphoreType.DMA,
        pltpu.SemaphoreType.DMA,
        pltpu.SemaphoreType.DMA,
        pltpu.SemaphoreType.DMA,
        pltpu.SemaphoreType.DMA,
        pltpu.SemaphoreType.DMA,
        pltpu.SemaphoreType.DMA,
        pltpu.SemaphoreType.DMA,
    ],
)
def _sc_gather(emb, tb, tc, ftab, t_flat, out, t_v, idx_e, idx_o, ftab_v,
               ob0, ob1, ob2, cb0, cb1, cb2, a0, a1, a2, b0, b1, b2,
               c0, c1, c2):
    _sc_body(emb, tb, tc, ftab, t_flat, out, t_v, idx_e, idx_o, ftab_v,
             (ob0, ob1, ob2), (cb0, cb1, cb2), (a0, a1, a2), (b0, b1, b2),
             (c0, c1, c2))


def kernel(t, embed_table):
    tb, tc, ftab = _build_tables(embed_table)
    return _sc_gather(embed_table, tb, tc, ftab, t.reshape(-1))


# speedup vs baseline: 1.0934x; 1.0934x over previous
import functools
import math

import jax
import jax.numpy as jnp
from jax import lax
from jax.experimental import pallas as pl
from jax.experimental.pallas import tpu as pltpu
from jax.experimental.pallas import tpu_sc as plsc

_V = 1024
_D = 256
_NF = 10
_W = _D + 2 * _NF
_B = 16384
_F = 3
_NW = 32
_ROWS_W = _B // _NW
_R = 32
_NCHUNK = _ROWS_W // _R
_NBUF = 3
_RT = 4 * _NF


def _fourier(rows):
    d = lax.broadcasted_iota(jnp.int32, (rows, 2 * _NF), 0).astype(jnp.float32)
    k = lax.broadcasted_iota(jnp.int32, (rows, 2 * _NF), 1)
    kk = k % _NF
    coef = (jnp.float32(math.pi) / jnp.float32(_V)) * (
        lax.shift_left(jnp.int32(1), kk).astype(jnp.float32))
    raw = coef * d
    return jnp.where(k < _NF, jnp.sin(raw), jnp.cos(raw))


def _tables_body(emb_ref, tb_ref, four_ref):
    tb_ref[...] = jnp.concatenate(
        [jnp.zeros((_V, 2 * _NF), jnp.float32), emb_ref[:, :_D - 2 * _NF]],
        axis=1)
    four_ref[...] = _fourier(_V)


def _build_tables(embed_table):
    return pl.pallas_call(
        _tables_body,
        out_shape=(
            jax.ShapeDtypeStruct((_V, _D), jnp.float32),
            jax.ShapeDtypeStruct((_V, 2 * _NF), jnp.float32),
        ),
    )(embed_table)


def _sc_body(emb, tb, rtab, t_flat, out, t_v, idx_e, idx_o, rtab_v,
             obufs, sas, sbs, sos):
    wid = lax.axis_index("s") * 2 + lax.axis_index("c")
    pltpu.sync_copy(t_flat.at[pl.ds(wid * (_ROWS_W * _F), _ROWS_W * _F)], t_v)
    pltpu.sync_copy(rtab, rtab_v)

    lane = lax.iota(jnp.int32, 16)
    zero = lane * 0
    for u in range(_ROWS_W // 16):
        b = lane + (u * 16)
        lo = b * _F
        t0 = plsc.load_gather(t_v, [lo])
        t1 = plsc.load_gather(t_v, [lo + 1])
        t2 = plsc.load_gather(t_v, [lo + 2])
        cc = u // (_R // 16)
        off = (u % (_R // 16)) * 16
        idx_e[cc, pl.ds(off, 16)] = t1 - t0
        idx_o[cc, pl.ds(off, 16)] = t2 - t1

    orow_base = wid * _ROWS_W

    def _fire(c):
        p = c % _NBUF
        ga = pltpu.async_copy(
            emb.at[idx_e.at[c]], obufs[p].at[:, pl.ds(0, _D)], sas[p])
        gb = pltpu.async_copy(
            tb.at[idx_o.at[c]], obufs[p].at[:, pl.ds(_D, _D)], sbs[p])
        return (ga, gb)

    def _repair(c):
        p = c % _NBUF
        obuf = obufs[p]
        for s in range(_R // 16):
            rows = lane + (s * 16)
            f0 = idx_e[c, pl.ds(s * 16, 16)] * _RT + (2 * _NF)
            f1 = idx_o[c, pl.ds(s * 16, 16)] * _RT

            def f_fix(k, _):
                fl = f0 + k
                vals = plsc.load_gather(rtab_v, [fl >> 7, fl & 127])
                plsc.store_scatter(obuf, [rows, zero + (_D + k)], vals)
                return 0

            def t_fix(k, _):
                fl = f1 + k
                vals = plsc.load_gather(rtab_v, [fl >> 7, fl & 127])
                plsc.store_scatter(obuf, [rows, zero + (2 * _D + k)], vals)
                return 0

            lax.fori_loop(0, 2 * _NF, f_fix, 0)
            lax.fori_loop(0, _RT, t_fix, 0)

    gh = [None] * _NCHUNK
    oh = [None] * _NCHUNK
    for c in range(_NBUF - 1):
        gh[c] = _fire(c)
    for c in range(_NCHUNK):
        if c + _NBUF - 1 < _NCHUNK:
            if c >= 1:
                oh[c - 1].wait()
            gh[c + _NBUF - 1] = _fire(c + _NBUF - 1)
        for h in gh[c]:
            h.wait()
        _repair(c)
        p = c % _NBUF
        oh[c] = pltpu.make_async_copy(
            obufs[p], out.at[pl.ds(orow_base + c * _R, _R)], sos[p])
        oh[c].start()
    oh[_NCHUNK - 2].wait()
    oh[_NCHUNK - 1].wait()


@functools.partial(
    pl.kernel,
    out_type=jax.ShapeDtypeStruct((_B, 2 * _W), jnp.float32),
    mesh=plsc.VectorSubcoreMesh(core_axis_name="c", subcore_axis_name="s"),
    compiler_params=pltpu.CompilerParams(needs_layout_passes=False),
    scratch_types=[
        pltpu.VMEM((_ROWS_W * _F,), jnp.int32),
        pltpu.VMEM((_NCHUNK, _R), jnp.int32),
        pltpu.VMEM((_NCHUNK, _R), jnp.int32),
        pltpu.VMEM((_V * _RT // 128, 128), jnp.float32),
        pltpu.VMEM((_R, 2 * _W), jnp.float32),
        pltpu.VMEM((_R, 2 * _W), jnp.float32),
        pltpu.VMEM((_R, 2 * _W), jnp.float32),
        pltpu.SemaphoreType.DMA,
        pltpu.SemaphoreType.DMA,
        pltpu.SemaphoreType.DMA,
        pltpu.SemaphoreType.DMA,
        pltpu.SemaphoreType.DMA,
        pltpu.SemaphoreType.DMA,
        pltpu.SemaphoreType.DMA,
        pltpu.SemaphoreType.DMA,
        pltpu.SemaphoreType.DMA,
    ],
)
def _sc_gather(emb, tb, rtab, t_flat, out, t_v, idx_e, idx_o, rtab_v,
               ob0, ob1, ob2, a0, a1, a2, b0, b1, b2, o0, o1, o2):
    _sc_body(emb, tb, rtab, t_flat, out, t_v, idx_e, idx_o, rtab_v,
             (ob0, ob1, ob2), (a0, a1, a2), (b0, b1, b2), (o0, o1, o2))


def kernel(t, embed_table):
    tb, four = _build_tables(embed_table)
    rtab = jnp.concatenate(
        [embed_table[:, _D - 2 * _NF:], four], axis=1).reshape(
            _V * _RT // 128, 128)
    return _sc_gather(embed_table, tb, rtab, t.reshape(-1))


# speedup vs baseline: 1.1987x; 1.0963x over previous
import functools
import math

import jax
import jax.numpy as jnp
from jax import lax
from jax.experimental import pallas as pl
from jax.experimental.pallas import tpu as pltpu
from jax.experimental.pallas import tpu_sc as plsc

_V = 1024
_D = 256
_NF = 10
_W = _D + 2 * _NF
_B = 16384
_F = 3
_NW = 32
_ROWS_W = _B // _NW
_R = 64
_NCHUNK = _ROWS_W // _R
_NBUF = 2
_RT = 4 * _NF


def _fourier(rows):
    d = lax.broadcasted_iota(jnp.int32, (rows, 2 * _NF), 0).astype(jnp.float32)
    k = lax.broadcasted_iota(jnp.int32, (rows, 2 * _NF), 1)
    kk = k % _NF
    coef = (jnp.float32(math.pi) / jnp.float32(_V)) * (
        lax.shift_left(jnp.int32(1), kk).astype(jnp.float32))
    raw = coef * d
    return jnp.where(k < _NF, jnp.sin(raw), jnp.cos(raw))


def _tables_body(emb_ref, tb_ref, four_ref):
    tb_ref[...] = jnp.concatenate(
        [jnp.zeros((_V, 2 * _NF), jnp.float32), emb_ref[:, :_D - 2 * _NF]],
        axis=1)
    four_ref[...] = _fourier(_V)


def _build_tables(embed_table):
    return pl.pallas_call(
        _tables_body,
        out_shape=(
            jax.ShapeDtypeStruct((_V, _D), jnp.float32),
            jax.ShapeDtypeStruct((_V, 2 * _NF), jnp.float32),
        ),
    )(embed_table)


def _sc_body(emb, tb, rtab, t_T, out, t_v, idx_e, idx_o, rtab_v,
             obufs, sas, sbs, sos):
    wid = lax.axis_index("s") * 2 + lax.axis_index("c")
    pltpu.sync_copy(t_T.at[:, pl.ds(wid * _ROWS_W, _ROWS_W)], t_v)
    pltpu.sync_copy(rtab, rtab_v)

    lane = lax.iota(jnp.int32, 16)
    zero = lane * 0
    for u in range(_ROWS_W // 16):
        off16 = u * 16
        t0 = t_v[0, pl.ds(off16, 16)]
        t1 = t_v[1, pl.ds(off16, 16)]
        t2 = t_v[2, pl.ds(off16, 16)]
        cc = u // (_R // 16)
        off = (u % (_R // 16)) * 16
        idx_e[cc, pl.ds(off, 16)] = t1 - t0
        idx_o[cc, pl.ds(off, 16)] = t2 - t1

    orow_base = wid * _ROWS_W

    def _fire(c):
        p = c % _NBUF
        ga = pltpu.async_copy(
            emb.at[idx_e.at[c]], obufs[p].at[:, pl.ds(0, _D)], sas[p])
        gb = pltpu.async_copy(
            tb.at[idx_o.at[c]], obufs[p].at[:, pl.ds(_D, _D)], sbs[p])
        return (ga, gb)

    def _repair(c):
        p = c % _NBUF
        obuf = obufs[p]
        for s in range(_R // 16):
            rows = lane + (s * 16)
            f0 = idx_e[c, pl.ds(s * 16, 16)] * _RT + (2 * _NF)
            f1 = idx_o[c, pl.ds(s * 16, 16)] * _RT

            def f_fix(k, _):
                fl = f0 + k
                vals = plsc.load_gather(rtab_v, [fl >> 7, fl & 127])
                plsc.store_scatter(obuf, [rows, zero + (_D + k)], vals)
                return 0

            def t_fix(k, _):
                fl = f1 + k
                vals = plsc.load_gather(rtab_v, [fl >> 7, fl & 127])
                plsc.store_scatter(obuf, [rows, zero + (2 * _D + k)], vals)
                return 0

            lax.fori_loop(0, 2 * _NF, f_fix, 0)
            lax.fori_loop(0, _RT, t_fix, 0)

    gh = [None] * _NCHUNK
    oh = [None] * _NCHUNK
    for c in range(_NBUF - 1):
        gh[c] = _fire(c)
    for c in range(_NCHUNK):
        if c + _NBUF - 1 < _NCHUNK:
            if c >= 1:
                oh[c - 1].wait()
            gh[c + _NBUF - 1] = _fire(c + _NBUF - 1)
        for h in gh[c]:
            h.wait()
        _repair(c)
        p = c % _NBUF
        oh[c] = pltpu.make_async_copy(
            obufs[p], out.at[pl.ds(orow_base + c * _R, _R)], sos[p])
        oh[c].start()
    oh[_NCHUNK - 2].wait()
    oh[_NCHUNK - 1].wait()


@functools.partial(
    pl.kernel,
    out_type=jax.ShapeDtypeStruct((_B, 2 * _W), jnp.float32),
    mesh=plsc.VectorSubcoreMesh(core_axis_name="c", subcore_axis_name="s"),
    compiler_params=pltpu.CompilerParams(needs_layout_passes=False),
    scratch_types=[
        pltpu.VMEM((_F, _ROWS_W), jnp.int32),
        pltpu.VMEM((_NCHUNK, _R), jnp.int32),
        pltpu.VMEM((_NCHUNK, _R), jnp.int32),
        pltpu.VMEM((_V * _RT // 128, 128), jnp.float32),
        pltpu.VMEM((_R, 2 * _W), jnp.float32),
        pltpu.VMEM((_R, 2 * _W), jnp.float32),
        pltpu.SemaphoreType.DMA,
        pltpu.SemaphoreType.DMA,
        pltpu.SemaphoreType.DMA,
        pltpu.SemaphoreType.DMA,
        pltpu.SemaphoreType.DMA,
        pltpu.SemaphoreType.DMA,
    ],
)
def _sc_gather(emb, tb, rtab, t_T, out, t_v, idx_e, idx_o, rtab_v,
               ob0, ob1, a0, a1, b0, b1, o0, o1):
    _sc_body(emb, tb, rtab, t_T, out, t_v, idx_e, idx_o, rtab_v,
             (ob0, ob1), (a0, a1), (b0, b1), (o0, o1))


def kernel(t, embed_table):
    tb, four = _build_tables(embed_table)
    rtab = jnp.concatenate(
        [embed_table[:, _D - 2 * _NF:], four], axis=1).reshape(
            _V * _RT // 128, 128)
    return _sc_gather(embed_table, tb, rtab, t.T)
